# baseline (speedup 1.0000x reference)
import functools

import jax
import jax.numpy as jnp
from jax import lax
from jax.experimental import pallas as pl
from jax.experimental.pallas import tpu as pltpu
from jax.experimental.pallas import tpu_sc as plsc

_EMB_D = 768
_E_POOL = 100
_E_P_LEN = 40
_B = 256
_HALF = _E_P_LEN // 2
_QB = 16
_STEPS = _B // _QB
_SEQ = 197
_XROWS = _SEQ * _B
_XB = _XROWS // _STEPS


def _route_body(xq_ref, ek_ref, idx_ref):
    ek = ek_ref[...]
    kn = ek / jnp.maximum(jnp.sqrt(jnp.sum(ek * ek, axis=1, keepdims=True)), 1e-12)
    xq = xq_ref[...]
    qn = xq / jnp.maximum(jnp.sqrt(jnp.sum(xq * xq, axis=1, keepdims=True)), 1e-12)
    s = lax.dot_general(
        qn, kn, (((1,), (1,)), ((), ())),
        preferred_element_type=jnp.float32,
    )
    m = jnp.max(s, axis=1, keepdims=True)
    col = lax.broadcasted_iota(jnp.int32, s.shape, 1)
    idx_ref[...] = jnp.min(jnp.where(s >= m, col, jnp.int32(2**30)), axis=1)


def _route(x_querry, e_k):
    return pl.pallas_call(
        _route_body,
        out_shape=jax.ShapeDtypeStruct((_B,), jnp.int32),
    )(x_querry, e_k)


def _gather_body(idx_ref, *refs):
    ep_refs = refs[:_QB]
    xb_ref = refs[_QB]
    ek_ref, ev_ref, xo_ref = refs[_QB + 1:_QB + 4]

    stacked = jnp.concatenate([r[...] for r in ep_refs], axis=0)
    swapped = jnp.swapaxes(stacked, 0, 1)
    ek_ref[...] = swapped[:_HALF]
    ev_ref[...] = swapped[_HALF:]
    xo_ref[...] = xb_ref[...]


def _gather(e_p, idx, xb_t):
    ep_spec = [
        pl.BlockSpec(
            (1, _E_P_LEN, _EMB_D),
            functools.partial(lambda j, b, idx_ref: (idx_ref[_QB * b + j], 0, 0), j),
        )
        for j in range(_QB)
    ]
    out_spec = pl.BlockSpec((_HALF, _QB, _EMB_D), lambda b, idx_ref: (0, b, 0))
    x_spec = pl.BlockSpec((_XB, _EMB_D), lambda b, idx_ref: (b, 0))
    return pl.pallas_call(
        _gather_body,
        grid_spec=pltpu.PrefetchScalarGridSpec(
            num_scalar_prefetch=1,
            grid=(_STEPS,),
            in_specs=ep_spec + [x_spec],
            out_specs=[out_spec, out_spec, x_spec],
        ),
        out_shape=[
            jax.ShapeDtypeStruct((_HALF, _B, _EMB_D), jnp.float32),
            jax.ShapeDtypeStruct((_HALF, _B, _EMB_D), jnp.float32),
            jax.ShapeDtypeStruct((_XROWS, _EMB_D), jnp.float32),
        ],
        compiler_params=pltpu.CompilerParams(
            dimension_semantics=("arbitrary",),
        ),
    )(idx, *([e_p] * _QB), xb_t)


def kernel(x_querry, l, x_block, e_p, e_k):
    del l
    idx = _route(x_querry, e_k)
    xb2 = jnp.swapaxes(x_block, 0, 1).reshape(_XROWS, _EMB_D)
    ek_t, ev_t, xo2 = _gather(e_p, idx, xb2)
    Ek = jnp.swapaxes(ek_t, 0, 1)
    Ev = jnp.swapaxes(ev_t, 0, 1)
    x_out = jnp.swapaxes(xo2.reshape(_SEQ, _B, _EMB_D), 0, 1)
    return (Ek, Ev, x_out)


# speedup vs baseline: 1.0025x; 1.0025x over previous
import functools

import jax
import jax.numpy as jnp
from jax import lax
from jax.experimental import pallas as pl
from jax.experimental.pallas import tpu as pltpu

_EMB_D = 768
_E_POOL = 100
_E_P_LEN = 40
_B = 256
_HALF = _E_P_LEN // 2
_QB = 16
_STEPS = _B // _QB
_SEQ = 197
_XROWS = _SEQ * _B
_XB = _XROWS // _STEPS


def _route_body(xq_ref, ek_ref, idx_ref):
    ek = ek_ref[...]
    kn = ek / jnp.maximum(jnp.sqrt(jnp.sum(ek * ek, axis=1, keepdims=True)), 1e-12)
    xq = xq_ref[...]
    qn = xq / jnp.maximum(jnp.sqrt(jnp.sum(xq * xq, axis=1, keepdims=True)), 1e-12)
    s = lax.dot_general(
        qn, kn, (((1,), (1,)), ((), ())),
        preferred_element_type=jnp.float32,
    )
    m = jnp.max(s, axis=1, keepdims=True)
    col = lax.broadcasted_iota(jnp.int32, s.shape, 1)
    idx_ref[...] = jnp.min(jnp.where(s >= m, col, jnp.int32(2**30)), axis=1)


def _route(x_querry, e_k):
    return pl.pallas_call(
        _route_body,
        out_shape=jax.ShapeDtypeStruct((_B,), jnp.int32),
    )(x_querry, e_k)


def _gather_body(idx_ref, *refs):
    ep_refs = refs[:_QB]
    xb_ref = refs[_QB]
    ek_ref, ev_ref, xo_ref = refs[_QB + 1:_QB + 4]

    stacked = jnp.concatenate([r[...] for r in ep_refs], axis=0)
    swapped = jnp.swapaxes(stacked, 0, 1)
    ek_ref[...] = swapped[:_HALF]
    ev_ref[...] = swapped[_HALF:]
    xo_ref[...] = xb_ref[...]


def _gather(e_p, idx, xb_t):
    ep_spec = [
        pl.BlockSpec(
            (1, _E_P_LEN, _EMB_D),
            functools.partial(lambda j, b, idx_ref: (idx_ref[_QB * b + j], 0, 0), j),
        )
        for j in range(_QB)
    ]
    out_spec = pl.BlockSpec((_HALF, _QB, _EMB_D), lambda b, idx_ref: (0, b, 0))
    x_spec = pl.BlockSpec((_XB, _EMB_D), lambda b, idx_ref: (b, 0))
    return pl.pallas_call(
        _gather_body,
        grid_spec=pltpu.PrefetchScalarGridSpec(
            num_scalar_prefetch=1,
            grid=(_STEPS,),
            in_specs=ep_spec + [x_spec],
            out_specs=[out_spec, out_spec, x_spec],
        ),
        out_shape=[
            jax.ShapeDtypeStruct((_HALF, _B, _EMB_D), jnp.float32),
            jax.ShapeDtypeStruct((_HALF, _B, _EMB_D), jnp.float32),
            jax.ShapeDtypeStruct((_XROWS, _EMB_D), jnp.float32),
        ],
        compiler_params=pltpu.CompilerParams(
            dimension_semantics=("arbitrary",),
        ),
    )(idx, *([e_p] * _QB), xb_t)


def kernel(x_querry, l, x_block, e_p, e_k):
    del l
    idx = _route(x_querry, e_k)
    xb2 = jnp.swapaxes(x_block, 0, 1).reshape(_XROWS, _EMB_D)
    ek_t, ev_t, xo2 = _gather(e_p, idx, xb2)
    Ek = jnp.swapaxes(ek_t, 0, 1)
    Ev = jnp.swapaxes(ev_t, 0, 1)
    x_out = jnp.swapaxes(xo2.reshape(_SEQ, _B, _EMB_D), 0, 1)
    return (Ek, Ev, x_out)
